# baseline (speedup 1.0000x reference)
import functools

import jax
import jax.numpy as jnp
from jax import lax
from jax.experimental import pallas as pl
from jax.experimental.pallas import tpu as pltpu
from jax.experimental.pallas import tpu_sc as plsc

_M = 16384
_D = 256
_K = 8192

_TM = 256
_TK = 2048
_NM = _M // _TM
_NK = _K // _TK

_NC = 2
_NS = 16
_NW = _NC * _NS
_BPW = _M // _NW
_CH = 128


def _assign_body(x2_ref, x_ref, ct_ref, c2_ref, idx_ref, val_ref,
                 bv_scr, bi_scr):
    k = pl.program_id(0)
    m = pl.program_id(1)

    xs = x_ref[...] * 2.0
    mm = jnp.dot(xs, ct_ref[...], preferred_element_type=jnp.float32)
    t = (x2_ref[...] - mm) + c2_ref[...]

    bmin = jnp.min(t, axis=1, keepdims=True)
    ids = lax.broadcasted_iota(jnp.int32, (_TM, _TK), 1)
    barg = jnp.min(jnp.where(t == bmin, ids, _K), axis=1, keepdims=True)
    gidx = barg + k * _TK

    sl = pl.ds(m * _TM, _TM)
    first = k == 0
    prev_v = jnp.where(first, jnp.inf, bv_scr[sl, :])
    prev_i = jnp.where(first, 0, bi_scr[sl, :])
    take = bmin < prev_v
    new_v = jnp.where(take, bmin, prev_v)
    new_i = jnp.where(take, gidx, prev_i)
    bv_scr[sl, :] = new_v
    bi_scr[sl, :] = new_i
    idx_ref[...] = new_i
    val_ref[...] = new_v


_assign_call = pl.pallas_call(
    _assign_body,
    grid=(_NK, _NM),
    in_specs=[
        pl.BlockSpec((_TM, 1), lambda k, m: (m, 0)),
        pl.BlockSpec((_TM, _D), lambda k, m: (m, 0)),
        pl.BlockSpec((_D, _TK), lambda k, m: (0, k)),
        pl.BlockSpec((1, _TK), lambda k, m: (0, k)),
    ],
    out_specs=[
        pl.BlockSpec((_TM, 1), lambda k, m: (m, 0)),
        pl.BlockSpec((_TM, 1), lambda k, m: (m, 0)),
    ],
    out_shape=[
        jax.ShapeDtypeStruct((_M, 1), jnp.int32),
        jax.ShapeDtypeStruct((_M, 1), jnp.float32),
    ],
    scratch_shapes=[
        pltpu.VMEM((_M, 1), jnp.float32),
        pltpu.VMEM((_M, 1), jnp.int32),
    ],
)


def _gather_body(table_hbm, idx_hbm, out_hbm, idx_v, rows_v, sem):
    wid = lax.axis_index("s") * _NC + lax.axis_index("c")
    base = wid * _BPW
    for j in range(_BPW // _CH):
        off = base + j * _CH
        pltpu.sync_copy(idx_hbm.at[pl.ds(off, _CH)], idx_v)
        pltpu.async_copy(table_hbm.at[idx_v], rows_v, sem).wait()
        pltpu.sync_copy(rows_v, out_hbm.at[pl.ds(off, _CH)])


@functools.lru_cache(maxsize=1)
def _make_gather_call():
    return functools.partial(
        pl.kernel,
        mesh=plsc.VectorSubcoreMesh(core_axis_name="c", subcore_axis_name="s"),
        out_type=jax.ShapeDtypeStruct((_M, _D), jnp.float32),
        scratch_types=[
            pltpu.VMEM((_CH,), jnp.int32),
            pltpu.VMEM((_CH, _D), jnp.float32),
            pltpu.SemaphoreType.DMA,
        ],
    )(_gather_body)


def kernel(x, codebook):
    orig_shape = x.shape
    flatten = x.reshape(-1, orig_shape[-1])
    x2 = jnp.sum(flatten ** 2, axis=1, keepdims=True)
    c2 = jnp.sum(codebook ** 2, axis=1)[None, :]
    ct = codebook.T

    idx2d, val2d = _assign_call(x2, flatten, ct, c2)
    embed_ind = idx2d.reshape(-1)
    commit_loss = jnp.sum(val2d) / (_M * _D)

    quantize = _make_gather_call()(codebook, embed_ind)
    quantize_st = quantize.reshape(orig_shape)
    return quantize_st, embed_ind.reshape(orig_shape[:-1]), commit_loss


# speedup vs baseline: 1.4935x; 1.4935x over previous
import functools

import jax
import jax.numpy as jnp
from jax import lax
from jax.experimental import pallas as pl
from jax.experimental.pallas import tpu as pltpu
from jax.experimental.pallas import tpu_sc as plsc

_M = 16384
_D = 256
_K = 8192

_TM = 256
_TK = 2048
_NM = _M // _TM
_NK = _K // _TK

_NC = 2
_NS = 16
_NW = _NC * _NS
_BPW = _M // _NW
_CH = 128


def _assign_body(x2_ref, x_ref, ct_ref, c2_ref, idx_ref, val_ref):
    xs = x_ref[...] * 2.0
    mm = jnp.dot(xs, ct_ref[...], preferred_element_type=jnp.float32)
    t = (x2_ref[...] - mm) + c2_ref[...]
    bmin = jnp.min(t, axis=1, keepdims=True)
    idsf = lax.broadcasted_iota(jnp.int32, (_TM, _K), 1).astype(jnp.float32)
    barg = jnp.min(jnp.where(t == bmin, idsf, float(_K)),
                   axis=1, keepdims=True)
    idx_ref[...] = barg.astype(jnp.int32)
    val_ref[...] = bmin


_assign_call = pl.pallas_call(
    _assign_body,
    grid=(_NM,),
    in_specs=[
        pl.BlockSpec((_TM, 1), lambda m: (m, 0)),
        pl.BlockSpec((_TM, _D), lambda m: (m, 0)),
        pl.BlockSpec((_D, _K), lambda m: (0, 0)),
        pl.BlockSpec((1, _K), lambda m: (0, 0)),
    ],
    out_specs=[
        pl.BlockSpec((_TM, 1), lambda m: (m, 0)),
        pl.BlockSpec((_TM, 1), lambda m: (m, 0)),
    ],
    out_shape=[
        jax.ShapeDtypeStruct((_M, 1), jnp.int32),
        jax.ShapeDtypeStruct((_M, 1), jnp.float32),
    ],
)


def _gather_body(table_hbm, idx_hbm, out_hbm, idx_v, rows_v, sem):
    wid = lax.axis_index("s") * _NC + lax.axis_index("c")
    base = wid * _BPW
    for j in range(_BPW // _CH):
        off = base + j * _CH
        pltpu.sync_copy(idx_hbm.at[pl.ds(off, _CH)], idx_v)
        pltpu.async_copy(table_hbm.at[idx_v], rows_v, sem).wait()
        pltpu.sync_copy(rows_v, out_hbm.at[pl.ds(off, _CH)])


@functools.lru_cache(maxsize=1)
def _make_gather_call():
    return functools.partial(
        pl.kernel,
        mesh=plsc.VectorSubcoreMesh(core_axis_name="c", subcore_axis_name="s"),
        out_type=jax.ShapeDtypeStruct((_M, _D), jnp.float32),
        scratch_types=[
            pltpu.VMEM((_CH,), jnp.int32),
            pltpu.VMEM((_CH, _D), jnp.float32),
            pltpu.SemaphoreType.DMA,
        ],
    )(_gather_body)


def kernel(x, codebook):
    orig_shape = x.shape
    flatten = x.reshape(-1, orig_shape[-1])
    x2 = jnp.sum(flatten ** 2, axis=1, keepdims=True)
    c2 = jnp.sum(codebook ** 2, axis=1)[None, :]
    ct = codebook.T

    idx2d, val2d = _assign_call(x2, flatten, ct, c2)
    embed_ind = idx2d.reshape(-1)
    commit_loss = jnp.sum(val2d) / (_M * _D)

    quantize = _make_gather_call()(codebook, embed_ind)
    quantize_st = quantize.reshape(orig_shape)
    return quantize_st, embed_ind.reshape(orig_shape[:-1]), commit_loss
